# Initial kernel scaffold; baseline (speedup 1.0000x reference)
import jax
import jax.numpy as jnp
from jax.experimental import pallas as pl


def kernel(x_user, weight):
    raise NotImplementedError("write your pallas kernel here")


# speedup vs baseline: 44.7753x; 44.7753x over previous
import functools

import jax
import jax.numpy as jnp
from jax import lax
from jax.experimental import pallas as pl
from jax.experimental.pallas import tpu as pltpu
from jax.experimental.pallas import tpu_sc as plsc

BATCH = 16384
HIST = 200
VOCAB = 500
DIM = 12

NCORES = 2
NSUB = 16
NWORK = NCORES * NSUB
LANES = 16

VPAD = 512
DPAD = 16

ROWS_PER_W = BATCH // NWORK
NBLK = ROWS_PER_W // LANES


@functools.partial(
    pl.kernel,
    out_type=jax.ShapeDtypeStruct((BATCH * DIM,), jnp.float32),
    mesh=plsc.VectorSubcoreMesh(core_axis_name="c", subcore_axis_name="s"),
    compiler_params=pltpu.CompilerParams(needs_layout_passes=False),
    scratch_types=[
        pltpu.VMEM((VPAD * DPAD,), jnp.float32),
        pltpu.VMEM((LANES * HIST,), jnp.int32),
        pltpu.VMEM((LANES * DIM,), jnp.float32),
    ],
)
def _embbag_kernel(x_hbm, w_hbm, out_hbm, w_v, x_v, o_v):
    wid = lax.axis_index("s") * NCORES + lax.axis_index("c")
    pltpu.sync_copy(w_hbm, w_v)

    iota = lax.broadcasted_iota(jnp.int32, (LANES,), 0)
    rowoff = iota * HIST
    inv_l = jnp.float32(1.0 / HIST)
    zero = jnp.zeros((LANES,), jnp.float32)

    def block_body(blk, carry):
        base = wid * ROWS_PER_W + blk * LANES
        pltpu.sync_copy(x_hbm.at[pl.ds(base * HIST, LANES * HIST)], x_v)

        def l_body(l, accs):
            xv = plsc.load_gather(x_v, [rowoff + l])
            pos = xv * DPAD
            return tuple(
                accs[d] + plsc.load_gather(w_v, [pos + d]) for d in range(DIM)
            )

        accs = lax.fori_loop(0, HIST, l_body, (zero,) * DIM)
        for d in range(DIM):
            plsc.store_scatter(o_v, [iota * DIM + d], accs[d] * inv_l)
        pltpu.sync_copy(o_v, out_hbm.at[pl.ds(base * DIM, LANES * DIM)])
        return carry

    lax.fori_loop(0, NBLK, block_body, 0)


def kernel(x_user, weight):
    xflat = x_user.reshape(-1).astype(jnp.int32)
    wpad = (
        jnp.zeros((VPAD, DPAD), jnp.float32)
        .at[:VOCAB, :DIM]
        .set(weight)
        .reshape(-1)
    )
    out = _embbag_kernel(xflat, wpad)
    return out.reshape(BATCH, DIM)


# speedup vs baseline: 46.0892x; 1.0293x over previous
import functools

import jax
import jax.numpy as jnp
from jax import lax
from jax.experimental import pallas as pl
from jax.experimental.pallas import tpu as pltpu
from jax.experimental.pallas import tpu_sc as plsc

BATCH = 16384
HIST = 200
VOCAB = 500
DIM = 12

NCORES = 2
NSUB = 16
NWORK = NCORES * NSUB
LANES = 16

VPAD = 512
DPAD = 16

ROWS_PER_W = BATCH // NWORK
NBLK = ROWS_PER_W // LANES
UNROLL = 4


@functools.partial(
    pl.kernel,
    out_type=jax.ShapeDtypeStruct((BATCH * DIM,), jnp.float32),
    mesh=plsc.VectorSubcoreMesh(core_axis_name="c", subcore_axis_name="s"),
    compiler_params=pltpu.CompilerParams(needs_layout_passes=False),
    scratch_types=[
        pltpu.VMEM((VPAD * DPAD,), jnp.float32),
        pltpu.VMEM((ROWS_PER_W * HIST,), jnp.int32),
        pltpu.VMEM((ROWS_PER_W * DIM,), jnp.float32),
    ],
)
def _embbag_kernel(x_hbm, w_hbm, out_hbm, w_v, x_v, o_v):
    wid = lax.axis_index("s") * NCORES + lax.axis_index("c")
    pltpu.sync_copy(w_hbm, w_v)
    pltpu.sync_copy(
        x_hbm.at[pl.ds(wid * ROWS_PER_W * HIST, ROWS_PER_W * HIST)], x_v
    )

    iota = lax.broadcasted_iota(jnp.int32, (LANES,), 0)
    rowoff = iota * HIST
    inv_l = jnp.float32(1.0 / HIST)
    zero = jnp.zeros((LANES,), jnp.float32)

    def block_body(blk, carry):
        xbase = rowoff + blk * (LANES * HIST)

        def l_body(j, accs):
            l0 = j * UNROLL
            new = list(accs)
            for u in range(UNROLL):
                xv = plsc.load_gather(x_v, [xbase + (l0 + u)])
                pos = xv * DPAD
                for d in range(DIM):
                    new[d] = new[d] + plsc.load_gather(w_v, [pos + d])
            return tuple(new)

        accs = lax.fori_loop(0, HIST // UNROLL, l_body, (zero,) * DIM)
        obase = iota * DIM + blk * (LANES * DIM)
        for d in range(DIM):
            plsc.store_scatter(o_v, [obase + d], accs[d] * inv_l)
        return carry

    lax.fori_loop(0, NBLK, block_body, 0)
    pltpu.sync_copy(o_v, out_hbm.at[pl.ds(wid * ROWS_PER_W * DIM, ROWS_PER_W * DIM)])


def kernel(x_user, weight):
    xflat = x_user.reshape(-1).astype(jnp.int32)
    wpad = (
        jnp.zeros((VPAD, DPAD), jnp.float32)
        .at[:VOCAB, :DIM]
        .set(weight)
        .reshape(-1)
    )
    out = _embbag_kernel(xflat, wpad)
    return out.reshape(BATCH, DIM)


# speedup vs baseline: 92.6049x; 2.0093x over previous
import functools

import jax
import jax.numpy as jnp
from jax import lax
from jax.experimental import pallas as pl
from jax.experimental.pallas import tpu as pltpu
from jax.experimental.pallas import tpu_sc as plsc

BATCH = 16384
HIST = 200
VOCAB = 500
DIM = 12

NCORES = 2
NSUB = 16
NWORK = NCORES * NSUB
LANES = 16

VPAD = 512
WSTRIDE = 17
HSTRIDE = 201

ROWS_PER_W = BATCH // NWORK
NBLK = ROWS_PER_W // LANES
UNROLL = 4


@functools.partial(
    pl.kernel,
    out_type=jax.ShapeDtypeStruct((BATCH * DIM,), jnp.float32),
    mesh=plsc.VectorSubcoreMesh(core_axis_name="c", subcore_axis_name="s"),
    compiler_params=pltpu.CompilerParams(needs_layout_passes=False),
    scratch_types=[
        pltpu.VMEM((VPAD * WSTRIDE,), jnp.float32),
        pltpu.VMEM((ROWS_PER_W * HSTRIDE,), jnp.int32),
        pltpu.VMEM((ROWS_PER_W * DIM,), jnp.float32),
    ],
)
def _embbag_kernel(x_hbm, w_hbm, out_hbm, w_v, x_v, o_v):
    wid = lax.axis_index("s") * NCORES + lax.axis_index("c")
    pltpu.sync_copy(w_hbm, w_v)
    pltpu.sync_copy(
        x_hbm.at[pl.ds(wid * ROWS_PER_W * HSTRIDE, ROWS_PER_W * HSTRIDE)], x_v
    )

    iota = lax.broadcasted_iota(jnp.int32, (LANES,), 0)
    rowoff = iota * HSTRIDE
    inv_l = jnp.float32(1.0 / HIST)
    zero = jnp.zeros((LANES,), jnp.float32)

    def block_body(blk, carry):
        xbase = rowoff + blk * (LANES * HSTRIDE)

        def l_body(j, accs):
            l0 = j * UNROLL
            new = list(accs)
            for u in range(UNROLL):
                xv = plsc.load_gather(x_v, [xbase + (l0 + u)])
                pos = xv * WSTRIDE
                for d in range(DIM):
                    new[d] = new[d] + plsc.load_gather(w_v, [pos + d])
            return tuple(new)

        accs = lax.fori_loop(0, HIST // UNROLL, l_body, (zero,) * DIM)
        obase = iota * DIM + blk * (LANES * DIM)
        for d in range(DIM):
            plsc.store_scatter(o_v, [obase + d], accs[d] * inv_l)
        return carry

    lax.fori_loop(0, NBLK, block_body, 0)
    pltpu.sync_copy(o_v, out_hbm.at[pl.ds(wid * ROWS_PER_W * DIM, ROWS_PER_W * DIM)])


def kernel(x_user, weight):
    xpad = jnp.pad(x_user.astype(jnp.int32), ((0, 0), (0, HSTRIDE - HIST)))
    wpad = (
        jnp.zeros((VPAD, WSTRIDE), jnp.float32)
        .at[:VOCAB, :DIM]
        .set(weight)
        .reshape(-1)
    )
    out = _embbag_kernel(xpad.reshape(-1), wpad)
    return out.reshape(BATCH, DIM)


# speedup vs baseline: 98.2790x; 1.0613x over previous
import functools

import jax
import jax.numpy as jnp
from jax import lax
from jax.experimental import pallas as pl
from jax.experimental.pallas import tpu as pltpu
from jax.experimental.pallas import tpu_sc as plsc

BATCH = 16384
HIST = 200
VOCAB = 500
DIM = 12

NCORES = 2
NSUB = 16
NWORK = NCORES * NSUB
LANES = 16

VPAD = 512
WSTRIDE = 17

ROWS_PER_W = BATCH // NWORK
NBLK = ROWS_PER_W // LANES
UNROLL = 4


@functools.partial(
    pl.kernel,
    out_type=jax.ShapeDtypeStruct((BATCH * DIM,), jnp.float32),
    mesh=plsc.VectorSubcoreMesh(core_axis_name="c", subcore_axis_name="s"),
    compiler_params=pltpu.CompilerParams(needs_layout_passes=False),
    scratch_types=[
        pltpu.VMEM((VPAD * WSTRIDE,), jnp.float32),
        pltpu.VMEM((ROWS_PER_W * HIST,), jnp.int32),
        pltpu.VMEM((ROWS_PER_W * DIM,), jnp.float32),
    ],
)
def _embbag_kernel(x_hbm, w_hbm, out_hbm, w_v, x_v, o_v):
    wid = lax.axis_index("s") * NCORES + lax.axis_index("c")
    pltpu.sync_copy(w_hbm, w_v)
    pltpu.sync_copy(
        x_hbm.at[pl.ds(wid * ROWS_PER_W * HIST, ROWS_PER_W * HIST)], x_v
    )

    iota = lax.broadcasted_iota(jnp.int32, (LANES,), 0)
    rowoff = iota * HIST
    inv_l = jnp.float32(1.0 / HIST)
    zero = jnp.zeros((LANES,), jnp.float32)

    def block_body(blk, carry):
        xbase = rowoff + iota + blk * (LANES * HIST)

        def l_body(j, accs):
            l0 = j * UNROLL
            new = list(accs)
            for u in range(UNROLL):
                off = xbase + (l0 + u)
                off = off - jnp.where(iota + (l0 + u) >= HIST, HIST, 0)
                xv = plsc.load_gather(x_v, [off])
                pos = xv * WSTRIDE
                for d in range(DIM):
                    new[d] = new[d] + plsc.load_gather(w_v, [pos + d])
            return tuple(new)

        accs = lax.fori_loop(0, HIST // UNROLL, l_body, (zero,) * DIM)
        obase = iota * DIM + blk * (LANES * DIM)
        for d in range(DIM):
            plsc.store_scatter(o_v, [obase + d], accs[d] * inv_l)
        return carry

    lax.fori_loop(0, NBLK, block_body, 0)
    pltpu.sync_copy(o_v, out_hbm.at[pl.ds(wid * ROWS_PER_W * DIM, ROWS_PER_W * DIM)])


def kernel(x_user, weight):
    xflat = x_user.reshape(-1).astype(jnp.int32)
    wpad = (
        jnp.zeros((VPAD, WSTRIDE), jnp.float32)
        .at[:VOCAB, :DIM]
        .set(weight)
        .reshape(-1)
    )
    out = _embbag_kernel(xflat, wpad)
    return out.reshape(BATCH, DIM)


# speedup vs baseline: 101.3320x; 1.0311x over previous
import functools

import jax
import jax.numpy as jnp
from jax import lax
from jax.experimental import pallas as pl
from jax.experimental.pallas import tpu as pltpu
from jax.experimental.pallas import tpu_sc as plsc

BATCH = 16384
HIST = 200
VOCAB = 500
DIM = 12

NCORES = 2
NSUB = 16
NWORK = NCORES * NSUB
LANES = 16

VPAD = 512
WSTRIDE = 17

ROWS_PER_W = BATCH // NWORK
NBLK = ROWS_PER_W // LANES
UNROLL = 4


@functools.partial(
    pl.kernel,
    out_type=jax.ShapeDtypeStruct((BATCH, DIM), jnp.float32),
    mesh=plsc.VectorSubcoreMesh(core_axis_name="c", subcore_axis_name="s"),
    compiler_params=pltpu.CompilerParams(
        needs_layout_passes=False, use_tc_tiling_on_sc=False
    ),
    scratch_types=[
        pltpu.VMEM((VPAD * WSTRIDE,), jnp.float32),
        pltpu.VMEM((ROWS_PER_W, HIST), jnp.int32),
        pltpu.VMEM((ROWS_PER_W, DIM), jnp.float32),
    ],
)
def _embbag_kernel(x_hbm, w_hbm, out_hbm, w_v, x_v, o_v):
    wid = lax.axis_index("s") * NCORES + lax.axis_index("c")
    pltpu.sync_copy(w_hbm, w_v)
    pltpu.sync_copy(x_hbm.at[pl.ds(wid * ROWS_PER_W, ROWS_PER_W)], x_v)

    iota = lax.broadcasted_iota(jnp.int32, (LANES,), 0)
    inv_l = jnp.float32(1.0 / HIST)
    zero = jnp.zeros((LANES,), jnp.float32)

    def block_body(blk, carry):
        rows = iota + blk * LANES

        def l_body(j, accs):
            l0 = j * UNROLL
            new = list(accs)
            for u in range(UNROLL):
                col = iota + (l0 + u)
                col = col - jnp.where(col >= HIST, HIST, 0)
                xv = plsc.load_gather(x_v, [rows, col])
                pos = xv * WSTRIDE
                for d in range(DIM):
                    new[d] = new[d] + plsc.load_gather(w_v, [pos + d])
            return tuple(new)

        accs = lax.fori_loop(0, HIST // UNROLL, l_body, (zero,) * DIM)
        for d in range(DIM):
            plsc.store_scatter(
                o_v, [rows, jnp.full((LANES,), d, jnp.int32)], accs[d] * inv_l
            )
        return carry

    lax.fori_loop(0, NBLK, block_body, 0)
    pltpu.sync_copy(o_v, out_hbm.at[pl.ds(wid * ROWS_PER_W, ROWS_PER_W)])


def kernel(x_user, weight):
    wpad = (
        jnp.zeros((VPAD, WSTRIDE), jnp.float32)
        .at[:VOCAB, :DIM]
        .set(weight)
        .reshape(-1)
    )
    return _embbag_kernel(x_user.astype(jnp.int32), wpad)


# speedup vs baseline: 115.8428x; 1.1432x over previous
import functools

import jax
import jax.numpy as jnp
from jax import lax
from jax.experimental import pallas as pl
from jax.experimental.pallas import tpu as pltpu
from jax.experimental.pallas import tpu_sc as plsc

BATCH = 16384
HIST = 200
VOCAB = 500
DIM = 12

NCORES = 2
NSUB = 16
NWORK = NCORES * NSUB
LANES = 16

VPAD = 512
NPAIR = DIM // 2
WSTRIDE = 7

ROWS_PER_W = BATCH // NWORK
NBLK = ROWS_PER_W // LANES
UNROLL = 4


@functools.partial(
    pl.kernel,
    out_type=jax.ShapeDtypeStruct((BATCH, DIM), jnp.float32),
    mesh=plsc.VectorSubcoreMesh(core_axis_name="c", subcore_axis_name="s"),
    compiler_params=pltpu.CompilerParams(
        needs_layout_passes=False, use_tc_tiling_on_sc=False
    ),
    scratch_types=[
        pltpu.VMEM((VPAD * WSTRIDE,), jnp.int32),
        pltpu.VMEM((ROWS_PER_W, HIST), jnp.int32),
        pltpu.VMEM((ROWS_PER_W, DIM), jnp.float32),
    ],
)
def _embbag_kernel(x_hbm, w_hbm, out_hbm, w_v, x_v, o_v):
    wid = lax.axis_index("s") * NCORES + lax.axis_index("c")
    pltpu.sync_copy(w_hbm, w_v)
    pltpu.sync_copy(x_hbm.at[pl.ds(wid * ROWS_PER_W, ROWS_PER_W)], x_v)

    iota = lax.broadcasted_iota(jnp.int32, (LANES,), 0)
    inv_l = jnp.float32(1.0 / HIST)
    zero = jnp.zeros((LANES,), jnp.float32)

    def block_body(blk, carry):
        rows = iota + blk * LANES

        def l_body(j, accs):
            l0 = j * UNROLL
            new = list(accs)
            for u in range(UNROLL):
                col = iota + (l0 + u)
                col = col - jnp.where(col >= HIST, HIST, 0)
                xv = plsc.load_gather(x_v, [rows, col])
                pos = xv * WSTRIDE
                for k in range(NPAIR):
                    wk = plsc.load_gather(w_v, [pos + k])
                    new[2 * k] = new[2 * k] + plsc.bitcast(
                        lax.shift_left(wk, 16), jnp.float32
                    )
                    new[2 * k + 1] = new[2 * k + 1] + plsc.bitcast(wk, jnp.float32)
            return tuple(new)

        accs = lax.fori_loop(0, HIST // UNROLL, l_body, (zero,) * DIM)
        for d in range(DIM):
            plsc.store_scatter(
                o_v, [rows, jnp.full((LANES,), d, jnp.int32)], accs[d] * inv_l
            )
        return carry

    lax.fori_loop(0, NBLK, block_body, 0)
    pltpu.sync_copy(o_v, out_hbm.at[pl.ds(wid * ROWS_PER_W, ROWS_PER_W)])


def kernel(x_user, weight):
    wb = jax.lax.bitcast_convert_type(
        weight.astype(jnp.bfloat16), jnp.uint16
    ).astype(jnp.uint32)
    packed = (wb[:, 0::2] | (wb[:, 1::2] << 16)).astype(jnp.int32)
    wpad = (
        jnp.zeros((VPAD, WSTRIDE), jnp.int32)
        .at[:VOCAB, :NPAIR]
        .set(packed)
        .reshape(-1)
    )
    return _embbag_kernel(x_user.astype(jnp.int32), wpad)


# speedup vs baseline: 133.7959x; 1.1550x over previous
import functools

import jax
import jax.numpy as jnp
from jax import lax
from jax.experimental import pallas as pl
from jax.experimental.pallas import tpu as pltpu
from jax.experimental.pallas import tpu_sc as plsc

BATCH = 16384
HIST = 200
VOCAB = 500
DIM = 12

NCORES = 2
NSUB = 16
NWORK = NCORES * NSUB
LANES = 16

VPAD = 512
NPAIR = DIM // 2
WSTRIDE = 7

ROWS_PER_W = BATCH // NWORK
CHUNK = 256
NCHUNK = ROWS_PER_W // CHUNK
BLK_PER_CHUNK = CHUNK // LANES
UNROLL = 4


@functools.partial(
    pl.kernel,
    out_type=jax.ShapeDtypeStruct((BATCH * DIM,), jnp.float32),
    mesh=plsc.VectorSubcoreMesh(core_axis_name="c", subcore_axis_name="s"),
    compiler_params=pltpu.CompilerParams(needs_layout_passes=False),
    scratch_types=[
        pltpu.VMEM((VPAD * WSTRIDE,), jnp.int32),
        pltpu.VMEM((CHUNK, HIST), jnp.int32),
        pltpu.VMEM((ROWS_PER_W * DIM,), jnp.float32),
    ],
)
def _embbag_kernel(x_hbm, w_hbm, out_hbm, w_v, x_v, o_v):
    wid = lax.axis_index("s") * NCORES + lax.axis_index("c")
    pltpu.sync_copy(w_hbm, w_v)

    iota = lax.broadcasted_iota(jnp.int32, (LANES,), 0)
    inv_l = jnp.float32(1.0 / HIST)
    zero = jnp.zeros((LANES,), jnp.float32)

    for chunk in range(NCHUNK):
        pltpu.sync_copy(
            x_hbm.at[pl.ds(wid * ROWS_PER_W + chunk * CHUNK, CHUNK)], x_v
        )

        def block_body(blk, carry):
            rows = iota + blk * LANES

            def l_body(j, accs):
                l0 = j * UNROLL
                new = list(accs)
                for u in range(UNROLL):
                    col = iota + (l0 + u)
                    col = col - jnp.where(col >= HIST, HIST, 0)
                    xv = plsc.load_gather(x_v, [rows, col])
                    pos = xv * WSTRIDE
                    for k in range(NPAIR):
                        wk = plsc.load_gather(w_v, [pos + k])
                        new[2 * k] = new[2 * k] + plsc.bitcast(
                            lax.shift_left(wk, 16), jnp.float32
                        )
                        new[2 * k + 1] = new[2 * k + 1] + plsc.bitcast(
                            wk, jnp.float32
                        )
                return tuple(new)

            accs = lax.fori_loop(0, HIST // UNROLL, l_body, (zero,) * DIM)
            obase = iota * DIM + (chunk * CHUNK * DIM) + blk * (LANES * DIM)
            for d in range(DIM):
                plsc.store_scatter(o_v, [obase + d], accs[d] * inv_l)
            return carry

        lax.fori_loop(0, BLK_PER_CHUNK, block_body, 0)

    pltpu.sync_copy(
        o_v, out_hbm.at[pl.ds(wid * ROWS_PER_W * DIM, ROWS_PER_W * DIM)]
    )


def kernel(x_user, weight):
    wb = jax.lax.bitcast_convert_type(
        weight.astype(jnp.bfloat16), jnp.uint16
    ).astype(jnp.uint32)
    packed = (wb[:, 0::2] | (wb[:, 1::2] << 16)).astype(jnp.int32)
    wpad = (
        jnp.zeros((VPAD, WSTRIDE), jnp.int32)
        .at[:VOCAB, :NPAIR]
        .set(packed)
        .reshape(-1)
    )
    out = _embbag_kernel(x_user.astype(jnp.int32), wpad)
    return out.reshape(BATCH, DIM)


# speedup vs baseline: 221.2837x; 1.6539x over previous
import functools

import jax
import jax.numpy as jnp
from jax import lax
from jax.experimental import pallas as pl
from jax.experimental.pallas import tpu as pltpu
from jax.experimental.pallas import tpu_sc as plsc

BATCH = 16384
HIST = 200
VOCAB = 500
DIM = 12

NCORES = 2
NSUB = 16
NWORK = NCORES * NSUB
LANES = 16

VPAD = 512
NPAIR = DIM // 2
WSTRIDE = 7

BAGS_PER_W = BATCH // NWORK
NBLK = BAGS_PER_W // LANES
UNROLL = 4


@functools.partial(
    pl.kernel,
    out_type=jax.ShapeDtypeStruct((DIM, BATCH), jnp.float32),
    mesh=plsc.VectorSubcoreMesh(core_axis_name="c", subcore_axis_name="s"),
    compiler_params=pltpu.CompilerParams(needs_layout_passes=False),
    scratch_types=[
        pltpu.VMEM((VPAD * WSTRIDE,), jnp.int32),
        pltpu.VMEM((HIST, BAGS_PER_W), jnp.int32),
        pltpu.VMEM((DIM, BAGS_PER_W), jnp.float32),
    ],
)
def _embbag_kernel(xt_hbm, w_hbm, out_hbm, w_v, x_v, o_v):
    wid = lax.axis_index("s") * NCORES + lax.axis_index("c")
    pltpu.sync_copy(w_hbm, w_v)
    pltpu.sync_copy(xt_hbm.at[:, pl.ds(wid * BAGS_PER_W, BAGS_PER_W)], x_v)

    inv_l = jnp.float32(1.0 / HIST)
    zero = jnp.zeros((LANES,), jnp.float32)

    def block_body(blk, carry):
        b0 = blk * LANES

        def l_body(j, accs):
            l0 = j * UNROLL
            new = list(accs)
            for u in range(UNROLL):
                xv = x_v[l0 + u, pl.ds(b0, LANES)]
                pos = xv * WSTRIDE
                for k in range(NPAIR):
                    wk = plsc.load_gather(w_v, [pos + k])
                    new[2 * k] = new[2 * k] + plsc.bitcast(
                        lax.shift_left(wk, 16), jnp.float32
                    )
                    new[2 * k + 1] = new[2 * k + 1] + plsc.bitcast(
                        wk, jnp.float32
                    )
            return tuple(new)

        accs = lax.fori_loop(0, HIST // UNROLL, l_body, (zero,) * DIM)
        for d in range(DIM):
            o_v[d, pl.ds(b0, LANES)] = accs[d] * inv_l
        return carry

    lax.fori_loop(0, NBLK, block_body, 0)
    pltpu.sync_copy(o_v, out_hbm.at[:, pl.ds(wid * BAGS_PER_W, BAGS_PER_W)])


def kernel(x_user, weight):
    wb = jax.lax.bitcast_convert_type(
        weight.astype(jnp.bfloat16), jnp.uint16
    ).astype(jnp.uint32)
    packed = (wb[:, 0::2] | (wb[:, 1::2] << 16)).astype(jnp.int32)
    wpad = (
        jnp.zeros((VPAD, WSTRIDE), jnp.int32)
        .at[:VOCAB, :NPAIR]
        .set(packed)
        .reshape(-1)
    )
    out_t = _embbag_kernel(x_user.astype(jnp.int32).T, wpad)
    return out_t.T
